# baseline (speedup 1.0000x reference)
import numpy as np
import jax
import jax.numpy as jnp
from jax.experimental import pallas as pl

NUM_Q = 4
DIM = 256
CDIM = 5

_LEVELS = np.array([8.0, 8.0, 8.0, 4.0, 4.0], dtype=np.float32)
_EPS = np.float32(1e-3)
_HALF_L = ((_LEVELS - 1.0) * (1.0 + _EPS) / 2.0).astype(np.float32)
_OFFSET = 0.5
_SHIFT = np.arctanh(_OFFSET / _HALF_L).astype(np.float32)
_HALF_W = np.floor(_LEVELS / 2.0).astype(np.float32)
_BASIS = np.concatenate([[1.0], np.cumprod(_LEVELS[:-1])]).astype(np.float32)
_SCALES = np.stack([(_LEVELS - 1.0) ** (-float(i)) for i in range(NUM_Q)]
                   ).astype(np.float32)
_CONSTS = np.concatenate([
    _SHIFT[None], _HALF_L[None], _BASIS[None], (_HALF_W * _BASIS)[None],
    (1.0 / _SCALES).astype(np.float32),
    (_SCALES / _HALF_W).astype(np.float32),
], axis=0).astype(np.float32)

_BLK = 1024


def _fsq_kernel(x_ref, w_in_ref, b_in_ref, w_out_ref, b_out_ref, c_ref,
                out_ref, idx_ref):
    c = c_ref[...]
    shift, half_l, basis, hw_basis = c[0:1], c[1:2], c[2:3], c[3:4]
    z = jnp.dot(x_ref[...], w_in_ref[...], preferred_element_type=jnp.float32)
    residual = z + b_in_ref[...]
    qsum = jnp.zeros_like(residual)
    idx_cols = []
    for qi in range(NUM_Q):
        inv_scale = c[4 + qi:5 + qi]
        qscale = c[8 + qi:9 + qi]
        zs = residual * inv_scale
        bounded = jnp.tanh(zs + shift) * half_l - _OFFSET
        lvl = jnp.round(bounded)
        quantized = lvl * qscale
        residual = residual - quantized
        qsum = qsum + quantized
        idx_f = jnp.sum(lvl * basis + hw_basis, axis=-1, keepdims=True)
        idx_cols.append(idx_f)
    out = jnp.dot(qsum, w_out_ref[...], preferred_element_type=jnp.float32)
    out_ref[...] = out + b_out_ref[...]
    idx_ref[...] = jnp.concatenate(idx_cols, axis=-1).astype(jnp.int32)


def kernel(x, W_in, b_in, W_out, b_out):
    B, N, _ = x.shape
    x2 = x.reshape(B * N, DIM)
    grid = (B * N) // _BLK
    out2, idx2 = pl.pallas_call(
        _fsq_kernel,
        grid=(grid,),
        in_specs=[
            pl.BlockSpec((_BLK, DIM), lambda i: (i, 0)),
            pl.BlockSpec((DIM, CDIM), lambda i: (0, 0)),
            pl.BlockSpec((1, CDIM), lambda i: (0, 0)),
            pl.BlockSpec((CDIM, DIM), lambda i: (0, 0)),
            pl.BlockSpec((1, DIM), lambda i: (0, 0)),
            pl.BlockSpec((12, CDIM), lambda i: (0, 0)),
        ],
        out_specs=[
            pl.BlockSpec((_BLK, DIM), lambda i: (i, 0)),
            pl.BlockSpec((_BLK, NUM_Q), lambda i: (i, 0)),
        ],
        out_shape=[
            jax.ShapeDtypeStruct((B * N, DIM), jnp.float32),
            jax.ShapeDtypeStruct((B * N, NUM_Q), jnp.int32),
        ],
    )(x2, W_in, b_in.reshape(1, CDIM), W_out, b_out.reshape(1, DIM),
      jnp.asarray(_CONSTS))
    return out2.reshape(B, N, DIM), idx2.reshape(B, N, NUM_Q)
